# baseline (speedup 1.0000x reference)
import functools

import jax
import jax.numpy as jnp
from jax import lax
from jax.experimental import pallas as pl
from jax.experimental.pallas import tpu as pltpu
from jax.experimental.pallas import tpu_sc as plsc

B = 4096
ED = 128
NU = 10
DEEP_IN = 3 * ED + NU
H1, H2, H3 = 1024, 512, 256
WIDE = 256

NC, NS = 2, 16
NW = NC * NS
B_PER_W = B // NW

BB = 1024
GRID = B // BB

_NT = (((1,), (1,)), ((), ()))


def _sc_gather_body(ut_hbm, st_hbm, ct_hbm, uid_hbm, sid_hbm, cid_hbm,
                    out_u, out_s, out_c,
                    idx_u, idx_s, idx_c, rows_u, rows_s, rows_c, sem):
    wid = lax.axis_index("s") * NC + lax.axis_index("c")
    base = wid * B_PER_W
    pltpu.sync_copy(uid_hbm.at[pl.ds(base, B_PER_W)], idx_u)
    pltpu.sync_copy(sid_hbm.at[pl.ds(base, B_PER_W)], idx_s)
    pltpu.sync_copy(cid_hbm.at[pl.ds(base, B_PER_W)], idx_c)
    g_u = pltpu.make_async_copy(ut_hbm.at[idx_u], rows_u, sem)
    g_s = pltpu.make_async_copy(st_hbm.at[idx_s], rows_s, sem)
    g_c = pltpu.make_async_copy(ct_hbm.at[idx_c], rows_c, sem)
    g_u.start()
    g_s.start()
    g_c.start()
    g_u.wait()
    g_s.wait()
    g_c.wait()
    pltpu.sync_copy(rows_u, out_u.at[pl.ds(base, B_PER_W)])
    pltpu.sync_copy(rows_s, out_s.at[pl.ds(base, B_PER_W)])
    pltpu.sync_copy(rows_c, out_c.at[pl.ds(base, B_PER_W)])


@functools.lru_cache(maxsize=1)
def _sc_gather_kernel():
    return pl.kernel(
        _sc_gather_body,
        out_type=[
            jax.ShapeDtypeStruct((B, ED), jnp.float32),
            jax.ShapeDtypeStruct((B, ED), jnp.float32),
            jax.ShapeDtypeStruct((B, ED), jnp.float32),
        ],
        mesh=plsc.VectorSubcoreMesh(core_axis_name="c",
                                    subcore_axis_name="s",
                                    num_cores=NC, num_subcores=NS),
        scratch_types=[
            pltpu.VMEM((B_PER_W,), jnp.int32),
            pltpu.VMEM((B_PER_W,), jnp.int32),
            pltpu.VMEM((B_PER_W,), jnp.int32),
            pltpu.VMEM((B_PER_W, ED), jnp.float32),
            pltpu.VMEM((B_PER_W, ED), jnp.float32),
            pltpu.VMEM((B_PER_W, ED), jnp.float32),
            pltpu.SemaphoreType.DMA,
        ],
    )


def _mlp_body(u_ref, s_ref, c_ref, n_ref, wide_ref,
              w1_ref, b1, w2_ref, b2, w3_ref, b3, wf1, wW, scal,
              out_ref, w1b, w2b, w3b):
    f32 = jnp.float32
    bf16 = jnp.bfloat16

    @pl.when(pl.program_id(0) == 0)
    def _cache_weights():
        w1b[:] = w1_ref[:].astype(bf16)
        w2b[:] = w2_ref[:].astype(bf16)
        w3b[:] = w3_ref[:].astype(bf16)

    def nt(a, b):
        return lax.dot_general(a, b, _NT, preferred_element_type=f32)

    h = nt(u_ref[:].astype(bf16), w1b[:, :ED])
    h += nt(s_ref[:].astype(bf16), w1b[:, ED:2 * ED])
    h += nt(c_ref[:].astype(bf16), w1b[:, 2 * ED:3 * ED])
    h += nt(n_ref[:].astype(bf16), w1b[:, 3 * ED:])
    h = jnp.maximum(h + b1[:], 0.0)
    h = jnp.maximum(nt(h.astype(bf16), w2b[:]) + b2[:], 0.0)
    h = jnp.maximum(nt(h.astype(bf16), w3b[:]) + b3[:], 0.0)
    wide_dot = jnp.sum(wide_ref[:] * wW[:], axis=1, keepdims=True)
    logit = nt(h, wf1[:]) + wide_dot + scal[0, 0]
    out_ref[:] = 1.0 / (1.0 + jnp.exp(-logit))


def _mlp_call(u_emb, s_emb, c_emb, num, wide_features,
              W1, b1, W2, b2, W3, b3, wf1, wWs, scal):
    def bspec(cols):
        return pl.BlockSpec((BB, cols), lambda i: (i, 0))

    def wspec(r, c):
        return pl.BlockSpec((r, c), lambda i: (0, 0))

    return pl.pallas_call(
        _mlp_body,
        grid=(GRID,),
        in_specs=[
            bspec(ED), bspec(ED), bspec(ED), bspec(NU), bspec(WIDE),
            wspec(H1, DEEP_IN), wspec(1, H1), wspec(H2, H1),
            wspec(1, H2), wspec(H3, H2), wspec(1, H3), wspec(1, H3),
            wspec(1, WIDE),
            pl.BlockSpec(memory_space=pltpu.SMEM),
        ],
        out_specs=pl.BlockSpec((BB, 1), lambda i: (i, 0)),
        out_shape=jax.ShapeDtypeStruct((B, 1), jnp.float32),
        scratch_shapes=[
            pltpu.VMEM((H1, DEEP_IN), jnp.bfloat16),
            pltpu.VMEM((H2, H1), jnp.bfloat16),
            pltpu.VMEM((H3, H2), jnp.bfloat16),
        ],
        compiler_params=pltpu.CompilerParams(
            dimension_semantics=("arbitrary",)),
    )(u_emb, s_emb, c_emb, num, wide_features,
      W1, b1, W2, b2, W3, b3, wf1, wWs, scal)


def kernel(wide_features, user_ids, shop_ids, category_ids,
           numerical_features, wide_W, wide_b, user_table, shop_table,
           cat_table, W1, b1, W2, b2, W3, b3, Wf, bf):
    uid = user_ids.astype(jnp.int32)
    sid = shop_ids.astype(jnp.int32)
    cid = category_ids.astype(jnp.int32)

    u_emb, s_emb, c_emb = _sc_gather_kernel()(
        user_table, shop_table, cat_table, uid, sid, cid)

    wf1 = Wf[:, 1:]
    wWs = wide_W * Wf[0, 0]
    cb = (bf + wide_b * Wf[0, 0]).reshape(1, 1)

    return _mlp_call(
        u_emb, s_emb, c_emb, numerical_features, wide_features,
        W1, b1.reshape(1, H1), W2, b2.reshape(1, H2),
        W3, b3.reshape(1, H3), wf1, wWs, cb)


# speedup vs baseline: 1.0685x; 1.0685x over previous
import functools

import jax
import jax.numpy as jnp
from jax import lax
from jax.experimental import pallas as pl
from jax.experimental.pallas import tpu as pltpu
from jax.experimental.pallas import tpu_sc as plsc

B = 4096
ED = 128
NU = 10
DEEP_IN = 3 * ED + NU
H1, H2, H3 = 1024, 512, 256
WIDE = 256

NC, NS = 2, 16
NW = NC * NS
B_PER_W = B // NW

BB = 1024
GRID = B // BB

_NT = (((1,), (1,)), ((), ()))


def _sc_gather_body(ut_hbm, st_hbm, ct_hbm, uid_hbm, sid_hbm, cid_hbm,
                    out_u, out_s, out_c,
                    idx_u, idx_s, idx_c, rows_u, rows_s, rows_c, sem):
    wid = lax.axis_index("s") * NC + lax.axis_index("c")
    base = wid * B_PER_W
    pltpu.sync_copy(uid_hbm.at[pl.ds(base, B_PER_W)], idx_u)
    pltpu.sync_copy(sid_hbm.at[pl.ds(base, B_PER_W)], idx_s)
    pltpu.sync_copy(cid_hbm.at[pl.ds(base, B_PER_W)], idx_c)
    g_u = pltpu.make_async_copy(ut_hbm.at[idx_u], rows_u, sem)
    g_s = pltpu.make_async_copy(st_hbm.at[idx_s], rows_s, sem)
    g_c = pltpu.make_async_copy(ct_hbm.at[idx_c], rows_c, sem)
    g_u.start()
    g_s.start()
    g_c.start()
    g_u.wait()
    g_s.wait()
    g_c.wait()
    pltpu.sync_copy(rows_u, out_u.at[pl.ds(base, B_PER_W)])
    pltpu.sync_copy(rows_s, out_s.at[pl.ds(base, B_PER_W)])
    pltpu.sync_copy(rows_c, out_c.at[pl.ds(base, B_PER_W)])


@functools.lru_cache(maxsize=1)
def _sc_gather_kernel():
    return pl.kernel(
        _sc_gather_body,
        out_type=[
            jax.ShapeDtypeStruct((B, ED), jnp.float32),
            jax.ShapeDtypeStruct((B, ED), jnp.float32),
            jax.ShapeDtypeStruct((B, ED), jnp.float32),
        ],
        mesh=plsc.VectorSubcoreMesh(core_axis_name="c",
                                    subcore_axis_name="s",
                                    num_cores=NC, num_subcores=NS),
        scratch_types=[
            pltpu.VMEM((B_PER_W,), jnp.int32),
            pltpu.VMEM((B_PER_W,), jnp.int32),
            pltpu.VMEM((B_PER_W,), jnp.int32),
            pltpu.VMEM((B_PER_W, ED), jnp.float32),
            pltpu.VMEM((B_PER_W, ED), jnp.float32),
            pltpu.VMEM((B_PER_W, ED), jnp.float32),
            pltpu.SemaphoreType.DMA,
        ],
    )


K1 = 400


def _mlp_body(u_ref, s_ref, c_ref, n_ref, wide_ref,
              w1_ref, b1c, w2_ref, b2, w3_ref, b3, wf1, wW, scal,
              out_ref, xcat, w1b, w2b, w3b):
    f32 = jnp.float32
    bf16 = jnp.bfloat16

    @pl.when(pl.program_id(0) == 0)
    def _cache_weights():
        w1b[:, :DEEP_IN] = w1_ref[:].astype(bf16)
        w1b[:, DEEP_IN:DEEP_IN + 1] = b1c[:].astype(bf16)
        w1b[:, DEEP_IN + 1:] = jnp.zeros((H1, K1 - DEEP_IN - 1), bf16)
        xcat[:, DEEP_IN:DEEP_IN + 1] = jnp.ones((BB, 1), bf16)
        xcat[:, DEEP_IN + 1:] = jnp.zeros((BB, K1 - DEEP_IN - 1), bf16)
        w2b[:] = w2_ref[:].astype(bf16)
        w3b[:] = w3_ref[:].astype(bf16)

    def nt(a, b):
        return lax.dot_general(a, b, _NT, preferred_element_type=f32)

    xcat[:, :ED] = u_ref[:].astype(bf16)
    xcat[:, ED:2 * ED] = s_ref[:].astype(bf16)
    xcat[:, 2 * ED:3 * ED] = c_ref[:].astype(bf16)
    xcat[:, 3 * ED:DEEP_IN] = n_ref[:].astype(bf16)
    h = jnp.maximum(nt(xcat[:], w1b[:]), 0.0)
    h = jnp.maximum(nt(h.astype(bf16), w2b[:]) + b2[:], 0.0)
    h = jnp.maximum(nt(h.astype(bf16), w3b[:]) + b3[:], 0.0)
    wide_dot = jnp.sum(wide_ref[:] * wW[:], axis=1, keepdims=True)
    logit = nt(h, wf1[:]) + wide_dot + scal[0, 0]
    out_ref[:] = 1.0 / (1.0 + jnp.exp(-logit))


def _mlp_call(u_emb, s_emb, c_emb, num, wide_features,
              W1, b1, W2, b2, W3, b3, wf1, wWs, scal):
    def bspec(cols):
        return pl.BlockSpec((BB, cols), lambda i: (i, 0))

    def wspec(r, c):
        return pl.BlockSpec((r, c), lambda i: (0, 0))

    return pl.pallas_call(
        _mlp_body,
        grid=(GRID,),
        in_specs=[
            bspec(ED), bspec(ED), bspec(ED), bspec(NU), bspec(WIDE),
            wspec(H1, DEEP_IN), wspec(H1, 1), wspec(H2, H1),
            wspec(1, H2), wspec(H3, H2), wspec(1, H3), wspec(1, H3),
            wspec(1, WIDE),
            pl.BlockSpec(memory_space=pltpu.SMEM),
        ],
        out_specs=pl.BlockSpec((BB, 1), lambda i: (i, 0)),
        out_shape=jax.ShapeDtypeStruct((B, 1), jnp.float32),
        scratch_shapes=[
            pltpu.VMEM((BB, K1), jnp.bfloat16),
            pltpu.VMEM((H1, K1), jnp.bfloat16),
            pltpu.VMEM((H2, H1), jnp.bfloat16),
            pltpu.VMEM((H3, H2), jnp.bfloat16),
        ],
        compiler_params=pltpu.CompilerParams(
            dimension_semantics=("arbitrary",)),
    )(u_emb, s_emb, c_emb, num, wide_features,
      W1, b1, W2, b2, W3, b3, wf1, wWs, scal)


def kernel(wide_features, user_ids, shop_ids, category_ids,
           numerical_features, wide_W, wide_b, user_table, shop_table,
           cat_table, W1, b1, W2, b2, W3, b3, Wf, bf):
    uid = user_ids.astype(jnp.int32)
    sid = shop_ids.astype(jnp.int32)
    cid = category_ids.astype(jnp.int32)

    u_emb, s_emb, c_emb = _sc_gather_kernel()(
        user_table, shop_table, cat_table, uid, sid, cid)

    wf1 = Wf[:, 1:]
    wWs = wide_W * Wf[0, 0]
    cb = (bf + wide_b * Wf[0, 0]).reshape(1, 1)

    return _mlp_call(
        u_emb, s_emb, c_emb, numerical_features, wide_features,
        W1, b1.reshape(H1, 1), W2, b2.reshape(1, H2),
        W3, b3.reshape(1, H3), wf1, wWs, cb)


# speedup vs baseline: 1.0868x; 1.0172x over previous
import functools

import jax
import jax.numpy as jnp
from jax import lax
from jax.experimental import pallas as pl
from jax.experimental.pallas import tpu as pltpu
from jax.experimental.pallas import tpu_sc as plsc

B = 4096
ED = 128
NU = 10
DEEP_IN = 3 * ED + NU
K1 = 400
H1, H2, H3 = 1024, 512, 256
WIDE = 256

NC, NS = 2, 16
NW = NC * NS
B_PER_W = B // NW

BB = 1024
GRID = B // BB

_NT = (((1,), (1,)), ((), ()))


def _sc_gather_body(ut_hbm, st_hbm, ct_hbm, uid_hbm, sid_hbm, cid_hbm,
                    numx_hbm, out_x,
                    idx_u, idx_s, idx_c, rows_u, rows_s, rows_c, rows_n,
                    sem):
    wid = lax.axis_index("s") * NC + lax.axis_index("c")
    base = wid * B_PER_W
    pltpu.sync_copy(uid_hbm.at[pl.ds(base, B_PER_W)], idx_u)
    pltpu.sync_copy(sid_hbm.at[pl.ds(base, B_PER_W)], idx_s)
    pltpu.sync_copy(cid_hbm.at[pl.ds(base, B_PER_W)], idx_c)
    g_u = pltpu.make_async_copy(ut_hbm.at[idx_u], rows_u, sem)
    g_s = pltpu.make_async_copy(st_hbm.at[idx_s], rows_s, sem)
    g_c = pltpu.make_async_copy(ct_hbm.at[idx_c], rows_c, sem)
    g_n = pltpu.make_async_copy(
        numx_hbm.at[pl.ds(base, B_PER_W)], rows_n, sem)
    g_u.start()
    g_s.start()
    g_c.start()
    g_n.start()
    g_u.wait()
    w_u = pltpu.make_async_copy(
        rows_u, out_x.at[pl.ds(base, B_PER_W), pl.ds(0, ED)], sem)
    w_u.start()
    g_s.wait()
    w_s = pltpu.make_async_copy(
        rows_s, out_x.at[pl.ds(base, B_PER_W), pl.ds(ED, ED)], sem)
    w_s.start()
    g_c.wait()
    w_c = pltpu.make_async_copy(
        rows_c, out_x.at[pl.ds(base, B_PER_W), pl.ds(2 * ED, ED)], sem)
    w_c.start()
    g_n.wait()
    w_n = pltpu.make_async_copy(
        rows_n, out_x.at[pl.ds(base, B_PER_W), pl.ds(3 * ED, K1 - 3 * ED)],
        sem)
    w_n.start()
    w_u.wait()
    w_s.wait()
    w_c.wait()
    w_n.wait()


@functools.lru_cache(maxsize=1)
def _sc_gather_kernel():
    return pl.kernel(
        _sc_gather_body,
        out_type=jax.ShapeDtypeStruct((B, K1), jnp.float32),
        mesh=plsc.VectorSubcoreMesh(core_axis_name="c",
                                    subcore_axis_name="s",
                                    num_cores=NC, num_subcores=NS),
        scratch_types=[
            pltpu.VMEM((B_PER_W,), jnp.int32),
            pltpu.VMEM((B_PER_W,), jnp.int32),
            pltpu.VMEM((B_PER_W,), jnp.int32),
            pltpu.VMEM((B_PER_W, ED), jnp.float32),
            pltpu.VMEM((B_PER_W, ED), jnp.float32),
            pltpu.VMEM((B_PER_W, ED), jnp.float32),
            pltpu.VMEM((B_PER_W, K1 - 3 * ED), jnp.float32),
            pltpu.SemaphoreType.DMA,
        ],
    )


def _mlp_body(x_ref, wide_ref, w1_ref, b1c, w2_ref, b2, w3_ref, b3,
              wf1, wW, scal, out_ref, w1b, w2b, w3b):
    f32 = jnp.float32
    bf16 = jnp.bfloat16

    @pl.when(pl.program_id(0) == 0)
    def _cache_weights():
        w1b[:, :DEEP_IN] = w1_ref[:].astype(bf16)
        w1b[:, DEEP_IN:DEEP_IN + 1] = b1c[:].astype(bf16)
        w1b[:, DEEP_IN + 1:] = jnp.zeros((H1, K1 - DEEP_IN - 1), bf16)
        w2b[:] = w2_ref[:].astype(bf16)
        w3b[:] = w3_ref[:].astype(bf16)

    def nt(a, b):
        return lax.dot_general(a, b, _NT, preferred_element_type=f32)

    h = jnp.maximum(nt(x_ref[:].astype(bf16), w1b[:]), 0.0)
    h = jnp.maximum(nt(h.astype(bf16), w2b[:]) + b2[:], 0.0)
    h = jnp.maximum(nt(h.astype(bf16), w3b[:]) + b3[:], 0.0)
    wide_dot = jnp.sum(wide_ref[:] * wW[:], axis=1, keepdims=True)
    logit = nt(h, wf1[:]) + wide_dot + scal[0, 0]
    out_ref[:] = 1.0 / (1.0 + jnp.exp(-logit))


def _mlp_call(x, wide_features, W1, b1c, W2, b2, W3, b3, wf1, wWs, scal):
    def bspec(cols):
        return pl.BlockSpec((BB, cols), lambda i: (i, 0))

    def wspec(r, c):
        return pl.BlockSpec((r, c), lambda i: (0, 0))

    return pl.pallas_call(
        _mlp_body,
        grid=(GRID,),
        in_specs=[
            bspec(K1), bspec(WIDE),
            wspec(H1, DEEP_IN), wspec(H1, 1), wspec(H2, H1),
            wspec(1, H2), wspec(H3, H2), wspec(1, H3), wspec(1, H3),
            wspec(1, WIDE),
            pl.BlockSpec(memory_space=pltpu.SMEM),
        ],
        out_specs=pl.BlockSpec((BB, 1), lambda i: (i, 0)),
        out_shape=jax.ShapeDtypeStruct((B, 1), jnp.float32),
        scratch_shapes=[
            pltpu.VMEM((H1, K1), jnp.bfloat16),
            pltpu.VMEM((H2, H1), jnp.bfloat16),
            pltpu.VMEM((H3, H2), jnp.bfloat16),
        ],
        compiler_params=pltpu.CompilerParams(
            dimension_semantics=("arbitrary",)),
    )(x, wide_features, W1, b1c, W2, b2, W3, b3, wf1, wWs, scal)


def kernel(wide_features, user_ids, shop_ids, category_ids,
           numerical_features, wide_W, wide_b, user_table, shop_table,
           cat_table, W1, b1, W2, b2, W3, b3, Wf, bf):
    uid = user_ids.astype(jnp.int32)
    sid = shop_ids.astype(jnp.int32)
    cid = category_ids.astype(jnp.int32)

    numx = jnp.concatenate(
        [numerical_features,
         jnp.ones((B, 1), jnp.float32),
         jnp.zeros((B, K1 - DEEP_IN - 1), jnp.float32)], axis=1)

    x = _sc_gather_kernel()(
        user_table, shop_table, cat_table, uid, sid, cid, numx)

    wf1 = Wf[:, 1:]
    wWs = wide_W * Wf[0, 0]
    cb = (bf + wide_b * Wf[0, 0]).reshape(1, 1)

    return _mlp_call(
        x, wide_features, W1, b1.reshape(H1, 1), W2, b2.reshape(1, H2),
        W3, b3.reshape(1, H3), wf1, wWs, cb)
